# baseline (speedup 1.0000x reference)
import jax
import jax.numpy as jnp
from jax.experimental import pallas as pl

NUM_CLASSES = 1000
N_IDX = 4096 * 20
W = 512
N_WIN = N_IDX * NUM_CLASSES // W
BLOCK_R = 4000
NUM_BLOCKS = N_WIN // BLOCK_R


def _one_hot_kernel(a0_ref, a1_ref, out_ref):
    a0 = a0_ref[0, 0, :]
    a1 = a1_ref[0, 0, :]
    c = jax.lax.broadcasted_iota(jnp.int32, (BLOCK_R, W), 1)
    hit = (c == a0[:, None]) | (c == a1[:, None])
    out_ref[...] = hit.astype(jnp.float32)


def kernel(inputs):
    idx = inputs.reshape(-1).astype(jnp.int32)
    hot = jnp.arange(N_IDX, dtype=jnp.int32) * NUM_CLASSES + idx
    r = jnp.arange(N_WIN, dtype=jnp.int32)
    i0 = (r * W) // NUM_CLASSES
    i1 = jnp.minimum(i0 + 1, N_IDX - 1)
    base = r * W
    a0 = (hot[i0] - base).reshape(NUM_BLOCKS, 1, BLOCK_R)
    a1 = (hot[i1] - base).reshape(NUM_BLOCKS, 1, BLOCK_R)

    out = pl.pallas_call(
        _one_hot_kernel,
        grid=(NUM_BLOCKS,),
        in_specs=[
            pl.BlockSpec((1, 1, BLOCK_R), lambda i: (i, 0, 0)),
            pl.BlockSpec((1, 1, BLOCK_R), lambda i: (i, 0, 0)),
        ],
        out_specs=pl.BlockSpec((BLOCK_R, W), lambda i: (i, 0)),
        out_shape=jax.ShapeDtypeStruct((N_WIN, W), jnp.float32),
    )(a0, a1)
    return out.reshape(4096, 20, NUM_CLASSES)


# speedup vs baseline: 2.2061x; 2.2061x over previous
import jax
import jax.numpy as jnp
from jax.experimental import pallas as pl

NUM_CLASSES = 1000
B0 = 4096
B1 = 20
BLOCK = 128
NUM_BLOCKS = B0 // BLOCK


def _one_hot_kernel(idx_ref, out_ref):
    idx = idx_ref[...]
    classes = jax.lax.broadcasted_iota(
        jnp.int32, (BLOCK, B1, NUM_CLASSES), 2)
    out_ref[...] = (idx[:, :, None] == classes).astype(jnp.float32)


def kernel(inputs):
    idx = inputs.astype(jnp.int32)
    out = pl.pallas_call(
        _one_hot_kernel,
        grid=(NUM_BLOCKS,),
        in_specs=[pl.BlockSpec((BLOCK, B1), lambda i: (i, 0))],
        out_specs=pl.BlockSpec((BLOCK, B1, NUM_CLASSES), lambda i: (i, 0, 0)),
        out_shape=jax.ShapeDtypeStruct((B0, B1, NUM_CLASSES), jnp.float32),
    )(idx)
    return out
